# baseline (speedup 1.0000x reference)
import jax
import jax.numpy as jnp
import numpy as np
from jax.experimental import pallas as pl

NUM_CH = 30
TIME_LEN = 3600
N_NODES = 1920
N_GRAPHS = 64
N_EDGES = 30720


def _pool_mat(L_in, L_out):
    M = np.zeros((L_in, L_out), dtype=np.float32)
    for i in range(L_out):
        s = (i * L_in) // L_out
        e = -((-(i + 1) * L_in) // L_out)
        M[s:e, i] = 1.0 / (e - s)
    return jnp.asarray(M)


_P199 = _pool_mat(199, 100)
_P1701 = _pool_mat(1701, 128)


def _dwconv(x, w, pad):
    return jax.lax.conv_general_dilated(
        x, w, (1,), [(pad, pad)], dimension_numbers=('NCH', 'OIH', 'NCH'),
        feature_group_count=NUM_CH)


def _seg_max(data, ids, n):
    out = jax.ops.segment_max(data, ids, num_segments=n)
    return jnp.where(jnp.isfinite(out), out, 0.0)


def _edgeconv(x, src, dst, w1, b1, w2, b2, g, bt):
    xi = x[dst]
    xj = x[src]
    h = jnp.concatenate([xi, xj - xi], axis=-1)
    h = jax.nn.relu(h @ w1 + b1)
    h = jax.nn.relu(h @ w2 + b2)
    h = h * g + bt
    return _seg_max(h, dst, x.shape[0])


def _mlp_kernel(p_ref, w1_ref, b1_ref, w2_ref, b2_ref, o_ref):
    h = jax.nn.relu(jnp.dot(p_ref[...], w1_ref[...],
                            preferred_element_type=jnp.float32) + b1_ref[...])
    o_ref[...] = jax.nn.relu(jnp.dot(h, w2_ref[...],
                                     preferred_element_type=jnp.float32) + b2_ref[...])


def kernel(x, edge_index, batch, w_conv1, bn1_g, bn1_b, w_conv2, bn2_g, bn2_b,
           m1_w1, m1_b1, m1_w2, m1_b2, m1_g, m1_bt,
           m2_w1, m2_b1, m2_w2, m2_b2, m2_g, m2_bt,
           m3_w1, m3_b1, m3_w2, m3_b2, m3_g, m3_bt,
           l1_w, l1_b, l2_w, l2_b):
    x = x.reshape(-1, NUM_CH, TIME_LEN)
    outs = []
    for s in range(18):
        seg = x[:, :, s * 200:(s + 1) * 200]
        y = _dwconv(seg, w_conv1, 15)
        y = y * bn1_g[None, :, None] + bn1_b[None, :, None]
        y = jax.nn.elu(y)
        outs.append(jnp.einsum('bcl,lo->bco', y, _P199))
    x = jnp.concatenate(outs, axis=2)
    y = _dwconv(x, w_conv2, 0)
    y = y * bn2_g[None, :, None] + bn2_b[None, :, None]
    y = jax.nn.elu(y)
    y = jnp.einsum('bcl,lo->bco', y, _P1701)
    nodes = y.reshape(-1, 128)
    src = edge_index[0]
    dst = edge_index[1]
    x1 = _edgeconv(nodes, src, dst, m1_w1, m1_b1, m1_w2, m1_b2, m1_g, m1_bt)
    p1 = _seg_max(x1, batch, N_GRAPHS)
    x2 = _edgeconv(x1, src, dst, m2_w1, m2_b1, m2_w2, m2_b2, m2_g, m2_bt)
    p2 = _seg_max(x2, batch, N_GRAPHS)
    x3 = _edgeconv(x2, src, dst, m3_w1, m3_b1, m3_w2, m3_b2, m3_g, m3_bt)
    p3 = _seg_max(x3, batch, N_GRAPHS)
    p = jnp.concatenate([p1, p2, p3], axis=1)
    out = pl.pallas_call(
        _mlp_kernel,
        out_shape=jax.ShapeDtypeStruct((N_GRAPHS, 128), jnp.float32),
    )(p, l1_w, l1_b, l2_w, l2_b)
    return out


# speedup vs baseline: 1.0842x; 1.0842x over previous
import functools

import jax
import jax.numpy as jnp
import numpy as np
from jax import lax
from jax.experimental import pallas as pl
from jax.experimental.pallas import tpu as pltpu
from jax.experimental.pallas import tpu_sc as plsc

NUM_CH = 30
TIME_LEN = 3600
N_NODES = 1920
N_GRAPHS = 64
N_EDGES = 30720
NEG = -3.0e38

_SC_NC = 2
_SC_NS = 16


def _pool_mat(L_in, L_out):
    M = np.zeros((L_in, L_out), dtype=np.float32)
    for i in range(L_out):
        s = (i * L_in) // L_out
        e = -((-(i + 1) * L_in) // L_out)
        M[s:e, i] = 1.0 / (e - s)
    return M


_P199 = _pool_mat(199, 100)
_P1701 = _pool_mat(1701, 128)


def _sc_gather(table, idx):
    V, K = table.shape
    B = idx.shape[0]
    nw = _SC_NC * _SC_NS
    b_per_w = B // nw
    chunk = min(b_per_w, max(8, (61440 // K) // 8 * 8))
    while b_per_w % chunk:
        chunk -= 8
    nch = b_per_w // chunk
    mesh = plsc.VectorSubcoreMesh(core_axis_name="c", subcore_axis_name="s")

    @functools.partial(
        pl.kernel,
        mesh=mesh,
        out_type=jax.ShapeDtypeStruct((B, K), jnp.float32),
        scratch_types=[
            pltpu.VMEM((chunk,), jnp.int32),
            pltpu.VMEM((chunk, K), jnp.float32),
            pltpu.SemaphoreType.DMA,
        ],
    )
    def gk(table_hbm, idx_hbm, out_hbm, idx_v, rows_v, sem):
        wid = lax.axis_index("s") * _SC_NC + lax.axis_index("c")
        base = wid * b_per_w
        for t in range(nch):
            off = base + t * chunk
            pltpu.sync_copy(idx_hbm.at[pl.ds(off, chunk)], idx_v)
            pltpu.async_copy(table_hbm.at[idx_v], rows_v, sem).wait()
            pltpu.sync_copy(rows_v, out_hbm.at[pl.ds(off, chunk)])

    return gk(table, idx)


def _elu(y):
    return jnp.where(y > 0, y, jnp.exp(y) - 1.0)


def _conv1_kernel(x_ref, w_ref, sb_ref, p_ref, o_ref):
    x = x_ref[...]
    R = x.shape[0]
    z15 = jnp.zeros((R, 15), jnp.float32)
    z16 = jnp.zeros((R, 16), jnp.float32)
    xp = jnp.concatenate([z15, x, z16], axis=1)
    acc = jnp.zeros((R, 199), jnp.float32)
    for k in range(32):
        acc = acc + xp[:, k:k + 199] * w_ref[:, k:k + 1]
    y = acc * sb_ref[:, 0:1] + sb_ref[:, 1:2]
    y = _elu(y)
    o_ref[...] = jnp.dot(y, p_ref[...], preferred_element_type=jnp.float32)


def _conv2_kernel(x_ref, tb_ref, sb_ref, p_ref, o_ref, y_ref):
    for j in range(14):
        xw = x_ref[0, :, j * 128:j * 128 + 256]
        yb = jnp.dot(xw, tb_ref[0], preferred_element_type=jnp.float32)
        yb = yb * sb_ref[0, :, 0:1] + sb_ref[0, :, 1:2]
        y_ref[:, j * 128:(j + 1) * 128] = _elu(yb)
    o_ref[0] = jnp.dot(y_ref[:, 0:1701], p_ref[...],
                       preferred_element_type=jnp.float32)


_SB = 120


def _mm_kernel(x_ref, m_ref, w_ref, b_ref, o_ref):
    for j in range(x_ref.shape[0] // _SB):
        sl = pl.ds(j * _SB, _SB)
        xj = x_ref[sl, :] * m_ref[sl, 0:1]
        o_ref[sl, :] = jnp.dot(
            xj, w_ref[...], preferred_element_type=jnp.float32) + b_ref[0:1, :]


def _segscan_max(h, ids, nrows):
    s = 1
    K = h.shape[1]
    while s < nrows:
        pid = jnp.concatenate(
            [jnp.full((s, 1), -2.0, jnp.float32), ids[:-s]], axis=0)
        ph = jnp.concatenate(
            [jnp.full((s, K), NEG, jnp.float32), h[:-s]], axis=0)
        h = jnp.where(ids == pid, jnp.maximum(h, ph), h)
        s *= 2
    return h


def _edge_kernel(g1_ref, g2_ref, ids_ref, w2_ref, vec_ref, o_ref, carry_ref):
    b = pl.program_id(0)
    K = o_ref.shape[1]

    @pl.when(b == 0)
    def _():
        carry_ref[0:1, :] = jnp.full((1, K), NEG, jnp.float32)
        carry_ref[1:2, :] = jnp.full((1, K), -1.0, jnp.float32)

    carh = carry_ref[0:1, :]
    carid = carry_ref[1:2, 0:1]
    for j in range(o_ref.shape[0] // _SB):
        sl = pl.ds(j * _SB, _SB)
        pre = jnp.maximum(g1_ref[sl, :] + g2_ref[sl, :], 0.0)
        h = jnp.dot(pre, w2_ref[...], preferred_element_type=jnp.float32)
        h = jnp.maximum(h + vec_ref[0:1, :], 0.0)
        h = h * vec_ref[1:2, :] + vec_ref[2:3, :]
        ids = ids_ref[sl, 0:1]
        h = jnp.where(ids == carid, jnp.maximum(h, carh), h)
        h = _segscan_max(h, ids, _SB)
        o_ref[sl, :] = h
        carh = h[_SB - 1:_SB, :]
        carid = ids[_SB - 1:_SB, :]
    carry_ref[0:1, :] = carh
    carry_ref[1:2, :] = jnp.broadcast_to(carid, (1, K))


def _pool_kernel(x_ref, m_ref, ids_ref, o_ref):
    K = o_ref.shape[1]
    carh = jnp.full((1, K), NEG, jnp.float32)
    carid = jnp.full((1, 1), -1.0, jnp.float32)
    for j in range(o_ref.shape[0] // _SB):
        sl = pl.ds(j * _SB, _SB)
        h = x_ref[sl, :] * m_ref[sl, 0:1]
        ids = ids_ref[sl, 0:1]
        h = jnp.where(ids == carid, jnp.maximum(h, carh), h)
        h = _segscan_max(h, ids, _SB)
        o_ref[sl, :] = h
        carh = h[_SB - 1:_SB, :]
        carid = ids[_SB - 1:_SB, :]


def _head_kernel(p_ref, m_ref, w1_ref, b1_ref, w2_ref, b2_ref, o_ref):
    p = p_ref[...] * m_ref[...][:, 0:1]
    h = jnp.maximum(
        jnp.dot(p, w1_ref[...], preferred_element_type=jnp.float32)
        + b1_ref[0:1, :], 0.0)
    o_ref[...] = jnp.maximum(
        jnp.dot(h, w2_ref[...], preferred_element_type=jnp.float32)
        + b2_ref[0:1, :], 0.0)


def _row8(v):
    return jnp.broadcast_to(v.astype(jnp.float32)[:, None], (v.shape[0], 8))


def _pad_idx(idx, tot):
    pad = tot - idx.shape[0]
    return jnp.concatenate([idx, jnp.zeros((pad,), jnp.int32)])


def kernel(x, edge_index, batch, w_conv1, bn1_g, bn1_b, w_conv2, bn2_g, bn2_b,
           m1_w1, m1_b1, m1_w2, m1_b2, m1_g, m1_bt,
           m2_w1, m2_b1, m2_w2, m2_b2, m2_g, m2_bt,
           m3_w1, m3_b1, m3_w2, m3_b2, m3_g, m3_bt,
           l1_w, l1_b, l2_w, l2_b):
    f32 = jnp.float32

    xs = x.reshape(N_NODES * 18, 200)
    wc1 = w_conv1[:, 0, :]
    w1r = jnp.tile(jnp.repeat(wc1, 18, axis=0), (N_GRAPHS, 1))
    sb1 = jnp.stack([jnp.repeat(jnp.tile(bn1_g, N_GRAPHS), 18),
                     jnp.repeat(jnp.tile(bn1_b, N_GRAPHS), 18)], axis=1)
    sb1 = jnp.pad(sb1, ((0, 0), (0, 6)))
    R1 = 192
    y1 = pl.pallas_call(
        _conv1_kernel,
        grid=(N_NODES * 18 // R1,),
        in_specs=[pl.BlockSpec((R1, 200), lambda i: (i, 0)),
                  pl.BlockSpec((R1, 32), lambda i: (i, 0)),
                  pl.BlockSpec((R1, 8), lambda i: (i, 0)),
                  pl.BlockSpec((199, 100), lambda i: (0, 0))],
        out_specs=pl.BlockSpec((R1, 100), lambda i: (i, 0)),
        out_shape=jax.ShapeDtypeStruct((N_NODES * 18, 100), f32),
    )(xs, w1r, sb1, jnp.asarray(_P199))
    x1800 = y1.reshape(N_NODES, 1800)

    wc2 = w_conv2[:, 0, :]
    di = np.arange(256)[:, None] - np.arange(128)[None, :]
    bandm = jnp.asarray((di >= 0) & (di < 100))
    tb_all = jnp.where(bandm[None],
                       wc2[:, jnp.asarray(np.clip(di, 0, 99))],
                       0.0)
    xc = x1800.reshape(N_GRAPHS, NUM_CH, 1800).transpose(1, 0, 2)
    xc = jnp.pad(xc, ((0, 0), (0, 0), (0, 120)))
    sb2 = jnp.stack([bn2_g, bn2_b], axis=1)[:, None, :]
    sb2 = jnp.pad(jnp.broadcast_to(sb2, (NUM_CH, N_GRAPHS, 2)),
                  ((0, 0), (0, 0), (0, 6)))
    nodes_cm = pl.pallas_call(
        _conv2_kernel,
        grid=(NUM_CH,),
        in_specs=[pl.BlockSpec((1, N_GRAPHS, 1920), lambda i: (i, 0, 0)),
                  pl.BlockSpec((1, 256, 128), lambda i: (i, 0, 0)),
                  pl.BlockSpec((1, N_GRAPHS, 8), lambda i: (i, 0, 0)),
                  pl.BlockSpec((1701, 128), lambda i: (0, 0))],
        out_specs=pl.BlockSpec((1, N_GRAPHS, 128), lambda i: (i, 0, 0)),
        out_shape=jax.ShapeDtypeStruct((NUM_CH, N_GRAPHS, 128), f32),
        scratch_shapes=[pltpu.VMEM((N_GRAPHS, 1792), f32)],
    )(xc, tb_all, sb2, jnp.asarray(_P1701))
    nodes = nodes_cm.transpose(1, 0, 2).reshape(N_NODES, 128)

    src = edge_index[0].astype(jnp.int32)
    dst = edge_index[1].astype(jnp.int32)
    perm = jnp.argsort(dst)
    dsts = dst[perm]
    srcs = src[perm]
    ids_e = _row8(dsts)
    ar_n = jnp.arange(N_NODES, dtype=jnp.int32)
    hi_e = jnp.searchsorted(dsts, ar_n, side='right').astype(jnp.int32)
    lo_e = jnp.searchsorted(dsts, ar_n, side='left').astype(jnp.int32)
    last_e = _pad_idx(jnp.clip(hi_e - 1, 0, N_EDGES - 1), 2048)
    mask_e = _row8((hi_e > lo_e).astype(f32))
    ones_n = jnp.ones((N_NODES, 8), f32)

    batch = batch.astype(jnp.int32)
    ids_b = _row8(batch)
    ar_g = jnp.arange(N_GRAPHS, dtype=jnp.int32)
    hi_g = jnp.searchsorted(batch, ar_g, side='right').astype(jnp.int32)
    lo_g = jnp.searchsorted(batch, ar_g, side='left').astype(jnp.int32)
    last_g = _pad_idx(jnp.clip(hi_g - 1, 0, N_NODES - 1), 256)
    mask_g = _row8((hi_g > lo_g).astype(f32))

    def edge_layer(xin, mask_in, w1, b1, w2, b2, g, bt):
        Kin = xin.shape[1]
        K = w1.shape[1]
        wt, wb = w1[:Kin], w1[Kin:]
        wcat = jnp.concatenate([wt - wb, wb], axis=1)
        bias = jnp.broadcast_to(
            jnp.concatenate([b1, jnp.zeros_like(b1)])[None], (8, 2 * K))
        uv = pl.pallas_call(
            _mm_kernel,
            out_shape=jax.ShapeDtypeStruct((N_NODES, 2 * K), f32),
        )(xin, mask_in, wcat, bias)
        g1 = _sc_gather(uv[:, :K], dsts)
        g2 = _sc_gather(uv[:, K:], srcs)
        vecs = jnp.concatenate([
            jnp.broadcast_to(b2[None], (1, K)),
            jnp.broadcast_to(g[None], (1, K)),
            jnp.broadcast_to(bt[None], (1, K)),
            jnp.zeros((5, K), f32)], axis=0)
        RB = 1920
        h = pl.pallas_call(
            _edge_kernel,
            grid=(N_EDGES // RB,),
            in_specs=[pl.BlockSpec((RB, K), lambda i: (i, 0)),
                      pl.BlockSpec((RB, K), lambda i: (i, 0)),
                      pl.BlockSpec((RB, 8), lambda i: (i, 0)),
                      pl.BlockSpec((K, K), lambda i: (0, 0)),
                      pl.BlockSpec((8, K), lambda i: (0, 0))],
            out_specs=pl.BlockSpec((RB, K), lambda i: (i, 0)),
            out_shape=jax.ShapeDtypeStruct((N_EDGES, K), f32),
            scratch_shapes=[pltpu.VMEM((8, K), f32)],
        )(g1, g2, ids_e, w2, vecs)
        return _sc_gather(h, last_e)[:N_NODES]

    def graph_pool(xl):
        K = xl.shape[1]
        scanned = pl.pallas_call(
            _pool_kernel,
            out_shape=jax.ShapeDtypeStruct((N_NODES, K), f32),
        )(xl, mask_e, ids_b)
        return _sc_gather(scanned, last_g)[:N_GRAPHS]

    x1 = edge_layer(nodes, ones_n, m1_w1, m1_b1, m1_w2, m1_b2, m1_g, m1_bt)
    p1 = graph_pool(x1)
    x2 = edge_layer(x1, mask_e, m2_w1, m2_b1, m2_w2, m2_b2, m2_g, m2_bt)
    p2 = graph_pool(x2)
    x3 = edge_layer(x2, mask_e, m3_w1, m3_b1, m3_w2, m3_b2, m3_g, m3_bt)
    p3 = graph_pool(x3)

    p = jnp.concatenate([p1, p2, p3], axis=1)
    out = pl.pallas_call(
        _head_kernel,
        out_shape=jax.ShapeDtypeStruct((N_GRAPHS, 128), f32),
    )(p, mask_g,
      l1_w, jnp.broadcast_to(l1_b[None], (8, 256)),
      l2_w, jnp.broadcast_to(l2_b[None], (8, 128)))
    return out
